# baseline (speedup 1.0000x reference)
import functools
import jax
import jax.numpy as jnp
from jax import lax
from jax.experimental import pallas as pl
from jax.experimental.pallas import tpu as pltpu
from jax.experimental.pallas import tpu_sc as plsc

N = 10000
E = 320000
D = 128
H = 128
ED = 16
OUT = 64
G = 16
EPS = 1e-5

NC = 2
NS = 16
NW = NC * NS
EPW = E // NW
C = 128
NFULL = EPW // C
TAIL = EPW - NFULL * C
SLAB = 632
SLAB_LAST = N - 15 * SLAB

_mesh = plsc.VectorSubcoreMesh(core_axis_name="c", subcore_axis_name="s",
                               num_cores=NC, num_subcores=NS)


def _sc_pre_body(attr_hbm, dst_hbm, zeros16_hbm, ones_hbm,
                 eagg_out, deg_out,
                 eagg_sh, deg_sh,
                 attrv0, attrv1, dstv0, dstv1, onesv,
                 tattrv, tdstv,
                 asem0, asem1, isem0, isem1, esem0, esem1, dsem0, dsem1):
    c = lax.axis_index("c")
    s = lax.axis_index("s")
    wid = c * NS + s
    base = wid * EPW
    r0 = s * SLAB

    attrb = (attrv0, attrv1)
    dstb = (dstv0, dstv1)
    asems = (asem0, asem1)
    isems = (isem0, isem1)
    esems = (esem0, esem1)
    dsems = (dsem0, dsem1)

    @pl.when(s < NS - 1)
    def _():
        pltpu.sync_copy(zeros16_hbm, eagg_sh.at[pl.ds(r0, SLAB)])
        pltpu.sync_copy(zeros16_hbm, deg_sh.at[pl.ds(r0, SLAB)])

    @pl.when(s == NS - 1)
    def _():
        pltpu.sync_copy(zeros16_hbm.at[pl.ds(0, SLAB_LAST)],
                        eagg_sh.at[pl.ds(r0, SLAB_LAST)])
        pltpu.sync_copy(zeros16_hbm.at[pl.ds(0, SLAB_LAST)],
                        deg_sh.at[pl.ds(r0, SLAB_LAST)])

    pltpu.sync_copy(ones_hbm, onesv)
    plsc.subcore_barrier()

    def issue_loads(b, i):
        eb = base + i * C
        pltpu.async_copy(attr_hbm.at[pl.ds(eb, C)], attrb[b], asems[b])
        pltpu.async_copy(dst_hbm.at[pl.ds(eb, C)], dstb[b], isems[b])

    def wait_load(b):
        pltpu.make_async_copy(attr_hbm.at[pl.ds(0, C)], attrb[b],
                              asems[b]).wait()
        pltpu.make_async_copy(dst_hbm.at[pl.ds(0, C)], dstb[b],
                              isems[b]).wait()

    def issue_scatters(b):
        pltpu.async_copy(attrb[b], eagg_sh.at[dstb[b]], esems[b], add=True)
        pltpu.async_copy(onesv, deg_sh.at[dstb[b]], dsems[b], add=True)

    def wait_scatters(b):
        pltpu.make_async_copy(attrb[b], eagg_sh.at[pl.ds(0, C)],
                              esems[b]).wait()
        pltpu.make_async_copy(onesv, deg_sh.at[pl.ds(0, C)],
                              dsems[b]).wait()

    issue_loads(0, 0)
    issue_loads(1, 1)

    def body(h, carry):
        ge = 2 * h
        wait_load(0)
        issue_scatters(0)
        wait_scatters(0)

        @pl.when(h < NFULL // 2 - 1)
        def _():
            issue_loads(0, ge + 2)

        wait_load(1)
        issue_scatters(1)
        wait_scatters(1)

        @pl.when(h < NFULL // 2 - 1)
        def _():
            issue_loads(1, ge + 3)

        return carry

    lax.fori_loop(0, NFULL // 2, body, 0)

    eb = base + NFULL * C
    pltpu.async_copy(attr_hbm.at[pl.ds(eb, TAIL)], tattrv, asem0).wait()
    pltpu.sync_copy(dst_hbm.at[pl.ds(eb, TAIL)], tdstv)
    pltpu.sync_copy(tattrv, eagg_sh.at[tdstv], add=True)
    pltpu.sync_copy(onesv.at[pl.ds(0, TAIL)], deg_sh.at[tdstv], add=True)
    plsc.subcore_barrier()

    @pl.when(s < NS - 1)
    def _():
        pltpu.sync_copy(eagg_sh.at[pl.ds(r0, SLAB)],
                        eagg_out.at[c, pl.ds(r0, SLAB)])
        pltpu.sync_copy(deg_sh.at[pl.ds(r0, SLAB)],
                        deg_out.at[c, pl.ds(r0, SLAB)])

    @pl.when(s == NS - 1)
    def _():
        pltpu.sync_copy(eagg_sh.at[pl.ds(r0, SLAB_LAST)],
                        eagg_out.at[c, pl.ds(r0, SLAB_LAST)])
        pltpu.sync_copy(deg_sh.at[pl.ds(r0, SLAB_LAST)],
                        deg_out.at[c, pl.ds(r0, SLAB_LAST)])


_sc_pre = functools.partial(
    pl.kernel,
    out_type=(jax.ShapeDtypeStruct((NC, N, ED), jnp.float32),
              jax.ShapeDtypeStruct((NC, N, ED), jnp.float32)),
    mesh=_mesh,
    compiler_params=pltpu.CompilerParams(use_tc_tiling_on_sc=False),
    scratch_types=(
        [pltpu.VMEM_SHARED((N, ED), jnp.float32)] * 2
        + [pltpu.VMEM((C, ED), jnp.float32)] * 2
        + [pltpu.VMEM((C,), jnp.int32)] * 2
        + [pltpu.VMEM((C, ED), jnp.float32)]
        + [pltpu.VMEM((TAIL, ED), jnp.float32)]
        + [pltpu.VMEM((TAIL,), jnp.int32)]
        + [pltpu.SemaphoreType.DMA] * 8
    ),
)(_sc_pre_body)


HW = H // NC
EPW2 = E // NS
NFULL2 = EPW2 // C
TAIL2 = EPW2 - NFULL2 * C
NPAIR2 = NFULL2 // 2


def _sc_agg_body(xh_hbm, src_hbm, dst_hbm, zeros_hbm,
                 out_hbm,
                 xh_sh, acc_sh,
                 rows0, rows1,
                 s00, s10,
                 d00, d10,
                 tsrcv, tdstv, trows,
                 isem0, jsem0, gsem0, ssem0, isem1, jsem1, gsem1, ssem1):
    c = lax.axis_index("c")
    s = lax.axis_index("s")
    base = s * EPW2
    r0 = s * SLAB
    col = c * HW

    srcb = (s00, s10)
    dstb = (d00, d10)
    rowsb = (rows0, rows1)
    isems = (isem0, isem1)
    jsems = (jsem0, jsem1)
    gsems = (gsem0, gsem1)
    ssems = (ssem0, ssem1)

    @pl.when(s < NS - 1)
    def _():
        pltpu.sync_copy(xh_hbm.at[pl.ds(r0, SLAB), pl.ds(col, HW)],
                        xh_sh.at[pl.ds(r0, SLAB)])
        pltpu.sync_copy(zeros_hbm, acc_sh.at[pl.ds(r0, SLAB)])

    @pl.when(s == NS - 1)
    def _():
        pltpu.sync_copy(xh_hbm.at[pl.ds(r0, SLAB_LAST), pl.ds(col, HW)],
                        xh_sh.at[pl.ds(r0, SLAB_LAST)])
        pltpu.sync_copy(zeros_hbm.at[pl.ds(0, SLAB_LAST)],
                        acc_sh.at[pl.ds(r0, SLAB_LAST)])

    def issue_idx(b, grp):
        eb = base + grp * C
        pltpu.async_copy(src_hbm.at[pl.ds(eb, C)], srcb[b], isems[b])
        pltpu.async_copy(dst_hbm.at[pl.ds(eb, C)], dstb[b], jsems[b])

    def wait_rows(sem):
        pltpu.make_async_copy(xh_sh.at[pl.ds(0, C)], rowsb[0], sem).wait()

    def wait_idx(sem):
        pltpu.make_async_copy(src_hbm.at[pl.ds(0, C)], srcb[0], sem).wait()

    def issue_gather(b):
        pltpu.async_copy(xh_sh.at[srcb[b]], rowsb[b], gsems[b])

    def issue_scatter(b):
        pltpu.async_copy(rowsb[b], acc_sh.at[dstb[b]], ssems[b], add=True)

    plsc.subcore_barrier()

    issue_idx(0, 0)
    issue_idx(1, 1)
    wait_idx(isems[0])
    issue_gather(0)

    def body(h, carry):
        ge = 2 * h
        wait_rows(gsems[0])
        wait_idx(jsems[0])
        issue_scatter(0)
        wait_idx(isems[1])
        issue_gather(1)
        wait_rows(ssems[0])

        @pl.when(h < NPAIR2 - 1)
        def _():
            issue_idx(0, ge + 2)

        wait_rows(gsems[1])
        wait_idx(jsems[1])
        issue_scatter(1)
        wait_rows(ssems[1])

        @pl.when(h < NPAIR2 - 1)
        def _():
            issue_idx(1, ge + 3)
            wait_idx(isems[0])
            issue_gather(0)

        return carry

    lax.fori_loop(0, NPAIR2, body, 0)

    eb = base + NFULL2 * C
    pltpu.sync_copy(src_hbm.at[pl.ds(eb, TAIL2)], tsrcv)
    pltpu.async_copy(xh_sh.at[tsrcv], trows, gsem0).wait()
    pltpu.sync_copy(dst_hbm.at[pl.ds(eb, TAIL2)], tdstv)
    pltpu.sync_copy(trows, acc_sh.at[tdstv], add=True)
    plsc.subcore_barrier()

    @pl.when(s < NS - 1)
    def _():
        pltpu.sync_copy(acc_sh.at[pl.ds(r0, SLAB)],
                        out_hbm.at[pl.ds(r0, SLAB), pl.ds(col, HW)])

    @pl.when(s == NS - 1)
    def _():
        pltpu.sync_copy(acc_sh.at[pl.ds(r0, SLAB_LAST)],
                        out_hbm.at[pl.ds(r0, SLAB_LAST), pl.ds(col, HW)])


_sc_agg = functools.partial(
    pl.kernel,
    out_type=jax.ShapeDtypeStruct((N, H), jnp.float32),
    mesh=_mesh,
    compiler_params=pltpu.CompilerParams(use_tc_tiling_on_sc=False),
    scratch_types=(
        [pltpu.VMEM_SHARED((N, HW), jnp.float32)] * 2
        + [pltpu.VMEM((C, HW), jnp.float32)] * 2
        + [pltpu.VMEM((C,), jnp.int32)] * 4
        + [pltpu.VMEM((TAIL2,), jnp.int32)] * 2
        + [pltpu.VMEM((TAIL2, HW), jnp.float32)]
        + [pltpu.SemaphoreType.DMA] * 8
    ),
)(_sc_agg_body)


_R = 1000
_GRID = N // _R


def _mm_body(x_ref, w_ref, b_ref, o_ref):
    o_ref[...] = (jnp.dot(x_ref[...], w_ref[...],
                          preferred_element_type=jnp.float32) + b_ref[...])


def _tc_mm(x, wt, b):
    return pl.pallas_call(
        _mm_body,
        grid=(_GRID,),
        in_specs=[
            pl.BlockSpec((_R, wt.shape[0]), lambda i: (i, 0)),
            pl.BlockSpec(wt.shape, lambda i: (0, 0)),
            pl.BlockSpec((1, wt.shape[1]), lambda i: (0, 0)),
        ],
        out_specs=pl.BlockSpec((_R, wt.shape[1]), lambda i: (i, 0)),
        out_shape=jax.ShapeDtypeStruct((N, wt.shape[1]), jnp.float32),
    )(x, wt, b)


def _post_body(sp_ref, xh_ref, eaggp_ref, degp_ref, wet_ref, be_ref,
               p_ref, st_ref, acc):
    eagg = eaggp_ref[0] + eaggp_ref[1]
    deg = degp_ref[0, :, 0:1] + degp_ref[1, :, 0:1]
    p = (sp_ref[...] + xh_ref[...]
         + jnp.dot(eagg, wet_ref[...], preferred_element_type=jnp.float32)
         + deg * be_ref[...])
    p_ref[...] = p

    @pl.when(pl.program_id(0) == 0)
    def _():
        acc[...] = jnp.zeros_like(acc)

    acc[0:1, :] += jnp.sum(p, axis=0, keepdims=True)
    acc[1:2, :] += jnp.sum(p * p, axis=0, keepdims=True)

    @pl.when(pl.program_id(0) == _GRID - 1)
    def _():
        st_ref[...] = acc[...]


def _tc_post(sp, xh, eaggp, degp, wet, be):
    return pl.pallas_call(
        _post_body,
        grid=(_GRID,),
        in_specs=[
            pl.BlockSpec((_R, H), lambda i: (i, 0)),
            pl.BlockSpec((_R, H), lambda i: (i, 0)),
            pl.BlockSpec((NC, _R, ED), lambda i: (0, i, 0)),
            pl.BlockSpec((NC, _R, ED), lambda i: (0, i, 0)),
            pl.BlockSpec((ED, H), lambda i: (0, 0)),
            pl.BlockSpec((1, H), lambda i: (0, 0)),
        ],
        out_specs=[
            pl.BlockSpec((_R, H), lambda i: (i, 0)),
            pl.BlockSpec((2, H), lambda i: (0, 0)),
        ],
        out_shape=[
            jax.ShapeDtypeStruct((N, H), jnp.float32),
            jax.ShapeDtypeStruct((2, H), jnp.float32),
        ],
        scratch_shapes=[pltpu.VMEM((2, H), jnp.float32)],
    )(sp, xh, eaggp, degp, wet, be)


def _bn_mm_body(p_ref, st_ref, g_ref, beta_ref, wt_ref, b_ref, o_ref):
    mu = st_ref[0:1, :] * (1.0 / N)
    var = st_ref[1:2, :] * (1.0 / N) - mu * mu
    xn = (p_ref[...] - mu) * lax.rsqrt(var + EPS) * g_ref[...] + beta_ref[...]
    h = jnp.maximum(xn, 0.0)
    o_ref[...] = (jnp.dot(h, wt_ref[...],
                          preferred_element_type=jnp.float32) + b_ref[...])


def _tc_bn_mm(p, st, g, beta, wt, b):
    return pl.pallas_call(
        _bn_mm_body,
        grid=(_GRID,),
        in_specs=[
            pl.BlockSpec((_R, H), lambda i: (i, 0)),
            pl.BlockSpec((2, H), lambda i: (0, 0)),
            pl.BlockSpec((1, H), lambda i: (0, 0)),
            pl.BlockSpec((1, H), lambda i: (0, 0)),
            pl.BlockSpec((H, H), lambda i: (0, 0)),
            pl.BlockSpec((1, H), lambda i: (0, 0)),
        ],
        out_specs=pl.BlockSpec((_R, H), lambda i: (i, 0)),
        out_shape=jax.ShapeDtypeStruct((N, H), jnp.float32),
    )(p, st, g, beta, wt, b)


def _final_body(p_ref, st_ref, g_ref, beta_ref, batch_ref, wfct_ref, bfc_ref,
                o_ref, accs, accc):
    mu = st_ref[0:1, :] * (1.0 / N)
    var = st_ref[1:2, :] * (1.0 / N) - mu * mu
    xn = (p_ref[...] - mu) * lax.rsqrt(var + EPS) * g_ref[...] + beta_ref[...]
    h = jnp.maximum(xn, 0.0)
    b = batch_ref[0, 0, :]
    oh = (b[:, None] == lax.broadcasted_iota(jnp.int32, (1, G), 1)
          ).astype(jnp.float32)

    @pl.when(pl.program_id(0) == 0)
    def _():
        accs[...] = jnp.zeros_like(accs)
        accc[...] = jnp.zeros_like(accc)

    dn = (((0,), (0,)), ((), ()))
    accs[...] += lax.dot_general(oh, h, dn,
                                 preferred_element_type=jnp.float32)
    accc[...] += lax.dot_general(oh, jnp.ones_like(h), dn,
                                 preferred_element_type=jnp.float32)

    @pl.when(pl.program_id(0) == _GRID - 1)
    def _():
        pooled = accs[...] / jnp.maximum(accc[...], 1.0)
        o_ref[...] = (jnp.dot(pooled, wfct_ref[...],
                              preferred_element_type=jnp.float32)
                      + bfc_ref[...])


def _tc_final(p, st, g, beta, batch3, wfct, bfc):
    return pl.pallas_call(
        _final_body,
        grid=(_GRID,),
        in_specs=[
            pl.BlockSpec((_R, H), lambda i: (i, 0)),
            pl.BlockSpec((2, H), lambda i: (0, 0)),
            pl.BlockSpec((1, H), lambda i: (0, 0)),
            pl.BlockSpec((1, H), lambda i: (0, 0)),
            pl.BlockSpec((1, 1, _R), lambda i: (i, 0, 0)),
            pl.BlockSpec((H, OUT), lambda i: (0, 0)),
            pl.BlockSpec((1, OUT), lambda i: (0, 0)),
        ],
        out_specs=pl.BlockSpec((G, OUT), lambda i: (0, 0)),
        out_shape=jax.ShapeDtypeStruct((G, OUT), jnp.float32),
        scratch_shapes=[pltpu.VMEM((G, H), jnp.float32),
                        pltpu.VMEM((G, H), jnp.float32)],
    )(p, st, g, beta, batch3, wfct, bfc)


def kernel(x, edge_attr, Wn1, bn1, We1, be1, Wn2, bn2, We2, be2,
           Wn3, bn3, We3, be3, g1, beta1, g2, beta2, g3, beta3,
           Wfc, bfc, edge_index, batch):
    f32 = jnp.float32
    src = edge_index[0].astype(jnp.int32)
    dst = edge_index[1].astype(jnp.int32)
    batch3 = batch.astype(jnp.int32).reshape(_GRID, 1, _R)

    zeros64 = jnp.zeros((SLAB, HW), f32)
    zeros16 = jnp.zeros((SLAB, ED), f32)
    ones16 = jnp.ones((C, ED), f32)

    def row(v):
        return v.reshape(1, -1).astype(f32)

    eaggp, degp = _sc_pre(edge_attr.astype(f32), dst, zeros16, ones16)

    xh1 = _tc_mm(x.astype(f32), Wn1.T.astype(f32), row(bn1))
    sp1 = _sc_agg(xh1, src, dst, zeros64)
    p1, st1 = _tc_post(sp1, xh1, eaggp, degp, We1.T.astype(f32), row(be1))

    xh2 = _tc_bn_mm(p1, st1, row(g1), row(beta1), Wn2.T.astype(f32), row(bn2))
    sp2 = _sc_agg(xh2, src, dst, zeros64)
    p2, st2 = _tc_post(sp2, xh2, eaggp, degp, We2.T.astype(f32), row(be2))

    xh3 = _tc_bn_mm(p2, st2, row(g2), row(beta2), Wn3.T.astype(f32), row(bn3))
    sp3 = _sc_agg(xh3, src, dst, zeros64)
    p3, st3 = _tc_post(sp3, xh3, eaggp, degp, We3.T.astype(f32), row(be3))

    return _tc_final(p3, st3, row(g3), row(beta3), batch3,
                     Wfc.T.astype(f32), row(bfc))


# speedup vs baseline: 1.4065x; 1.4065x over previous
import functools
import jax
import jax.numpy as jnp
from jax import lax
from jax.experimental import pallas as pl
from jax.experimental.pallas import tpu as pltpu
from jax.experimental.pallas import tpu_sc as plsc

N = 10000
E = 320000
D = 128
H = 128
ED = 16
OUT = 64
G = 16
EPS = 1e-5

NC = 2
NS = 16
NW = NC * NS
EPW = E // NW
C = 128
NFULL = EPW // C
TAIL = EPW - NFULL * C
SLAB = 632
SLAB_LAST = N - 15 * SLAB

_mesh = plsc.VectorSubcoreMesh(core_axis_name="c", subcore_axis_name="s",
                               num_cores=NC, num_subcores=NS)


def _sc_pre_body(attr_hbm, dst_hbm, zeros16_hbm, ones_hbm,
                 eagg_out, deg_out,
                 eagg_sh, deg_sh,
                 attrv0, attrv1, dstv0, dstv1, onesv,
                 tattrv, tdstv,
                 asem0, asem1, isem0, isem1, esem0, esem1, dsem0, dsem1):
    c = lax.axis_index("c")
    s = lax.axis_index("s")
    wid = c * NS + s
    base = wid * EPW
    r0 = s * SLAB

    attrb = (attrv0, attrv1)
    dstb = (dstv0, dstv1)
    asems = (asem0, asem1)
    isems = (isem0, isem1)
    esems = (esem0, esem1)
    dsems = (dsem0, dsem1)

    @pl.when(s < NS - 1)
    def _():
        pltpu.sync_copy(zeros16_hbm, eagg_sh.at[pl.ds(r0, SLAB)])
        pltpu.sync_copy(zeros16_hbm, deg_sh.at[pl.ds(r0, SLAB)])

    @pl.when(s == NS - 1)
    def _():
        pltpu.sync_copy(zeros16_hbm.at[pl.ds(0, SLAB_LAST)],
                        eagg_sh.at[pl.ds(r0, SLAB_LAST)])
        pltpu.sync_copy(zeros16_hbm.at[pl.ds(0, SLAB_LAST)],
                        deg_sh.at[pl.ds(r0, SLAB_LAST)])

    pltpu.sync_copy(ones_hbm, onesv)
    plsc.subcore_barrier()

    def issue_loads(b, i):
        eb = base + i * C
        pltpu.async_copy(attr_hbm.at[pl.ds(eb, C)], attrb[b], asems[b])
        pltpu.async_copy(dst_hbm.at[pl.ds(eb, C)], dstb[b], isems[b])

    def wait_load(b):
        pltpu.make_async_copy(attr_hbm.at[pl.ds(0, C)], attrb[b],
                              asems[b]).wait()
        pltpu.make_async_copy(dst_hbm.at[pl.ds(0, C)], dstb[b],
                              isems[b]).wait()

    def issue_scatters(b):
        pltpu.async_copy(attrb[b], eagg_sh.at[dstb[b]], esems[b], add=True)
        pltpu.async_copy(onesv, deg_sh.at[dstb[b]], dsems[b], add=True)

    def wait_scatters(b):
        pltpu.make_async_copy(attrb[b], eagg_sh.at[pl.ds(0, C)],
                              esems[b]).wait()
        pltpu.make_async_copy(onesv, deg_sh.at[pl.ds(0, C)],
                              dsems[b]).wait()

    issue_loads(0, 0)
    issue_loads(1, 1)

    def body(h, carry):
        ge = 2 * h
        wait_load(0)
        issue_scatters(0)
        wait_scatters(0)

        @pl.when(h < NFULL // 2 - 1)
        def _():
            issue_loads(0, ge + 2)

        wait_load(1)
        issue_scatters(1)
        wait_scatters(1)

        @pl.when(h < NFULL // 2 - 1)
        def _():
            issue_loads(1, ge + 3)

        return carry

    lax.fori_loop(0, NFULL // 2, body, 0)

    eb = base + NFULL * C
    pltpu.async_copy(attr_hbm.at[pl.ds(eb, TAIL)], tattrv, asem0).wait()
    pltpu.sync_copy(dst_hbm.at[pl.ds(eb, TAIL)], tdstv)
    pltpu.sync_copy(tattrv, eagg_sh.at[tdstv], add=True)
    pltpu.sync_copy(onesv.at[pl.ds(0, TAIL)], deg_sh.at[tdstv], add=True)
    plsc.subcore_barrier()

    @pl.when(s < NS - 1)
    def _():
        pltpu.sync_copy(eagg_sh.at[pl.ds(r0, SLAB)],
                        eagg_out.at[c, pl.ds(r0, SLAB)])
        pltpu.sync_copy(deg_sh.at[pl.ds(r0, SLAB)],
                        deg_out.at[c, pl.ds(r0, SLAB)])

    @pl.when(s == NS - 1)
    def _():
        pltpu.sync_copy(eagg_sh.at[pl.ds(r0, SLAB_LAST)],
                        eagg_out.at[c, pl.ds(r0, SLAB_LAST)])
        pltpu.sync_copy(deg_sh.at[pl.ds(r0, SLAB_LAST)],
                        deg_out.at[c, pl.ds(r0, SLAB_LAST)])


_sc_pre = functools.partial(
    pl.kernel,
    out_type=(jax.ShapeDtypeStruct((NC, N, ED), jnp.float32),
              jax.ShapeDtypeStruct((NC, N, ED), jnp.float32)),
    mesh=_mesh,
    compiler_params=pltpu.CompilerParams(use_tc_tiling_on_sc=False),
    scratch_types=(
        [pltpu.VMEM_SHARED((N, ED), jnp.float32)] * 2
        + [pltpu.VMEM((C, ED), jnp.float32)] * 2
        + [pltpu.VMEM((C,), jnp.int32)] * 2
        + [pltpu.VMEM((C, ED), jnp.float32)]
        + [pltpu.VMEM((TAIL, ED), jnp.float32)]
        + [pltpu.VMEM((TAIL,), jnp.int32)]
        + [pltpu.SemaphoreType.DMA] * 8
    ),
)(_sc_pre_body)


NB = 4
CA = 64
NFULLA = EPW // CA
TAILA = EPW - NFULLA * CA
NSTEADY = NFULLA // NB - 1
REM = NFULLA - NB * NSTEADY


def _sc_agg_body(xh_hbm, src_hbm, dst_hbm, zeros_hbm,
                 out_hbm,
                 agg_sh,
                 rows0, rows1, rows2, rows3,
                 s0, s1, s2, s3,
                 d0, d1, d2, d3,
                 tsrcv, tdstv, trows,
                 isem0, isem1, isem2, isem3,
                 jsem0, jsem1, jsem2, jsem3,
                 gsem0, gsem1, gsem2, gsem3,
                 ssem0, ssem1, ssem2, ssem3):
    c = lax.axis_index("c")
    s = lax.axis_index("s")
    wid = c * NS + s
    base = wid * EPW
    r0 = s * SLAB

    srcb = (s0, s1, s2, s3)
    dstb = (d0, d1, d2, d3)
    rowsb = (rows0, rows1, rows2, rows3)
    isems = (isem0, isem1, isem2, isem3)
    jsems = (jsem0, jsem1, jsem2, jsem3)
    gsems = (gsem0, gsem1, gsem2, gsem3)
    ssems = (ssem0, ssem1, ssem2, ssem3)

    @pl.when(s < NS - 1)
    def _():
        pltpu.sync_copy(zeros_hbm, agg_sh.at[pl.ds(r0, SLAB)])

    @pl.when(s == NS - 1)
    def _():
        pltpu.sync_copy(zeros_hbm.at[pl.ds(0, SLAB_LAST)],
                        agg_sh.at[pl.ds(r0, SLAB_LAST)])

    def issue_idx(b, ch):
        eb = base + ch * CA
        pltpu.async_copy(src_hbm.at[pl.ds(eb, CA)], srcb[b], isems[b])
        pltpu.async_copy(dst_hbm.at[pl.ds(eb, CA)], dstb[b], jsems[b])

    def wait_rows(sem):
        pltpu.make_async_copy(xh_hbm.at[pl.ds(0, CA)], rowsb[0], sem).wait()

    def wait_idx(sem):
        pltpu.make_async_copy(src_hbm.at[pl.ds(0, CA)], srcb[0], sem).wait()

    def issue_gather(b):
        pltpu.async_copy(xh_hbm.at[srcb[b]], rowsb[b], gsems[b])

    def issue_scatter(b):
        pltpu.async_copy(rowsb[b], agg_sh.at[dstb[b]], ssems[b], add=True)

    plsc.subcore_barrier()

    for b in range(NB):
        issue_idx(b, b)
    for b in range(NB):
        wait_idx(isems[b])
        issue_gather(b)

    def body(h, carry):
        ch0 = NB * h
        for k in range(NB):
            wait_rows(gsems[k])
            wait_idx(jsems[k])
            issue_scatter(k)
            issue_idx(k, ch0 + k + NB)
            wait_rows(ssems[k])
            wait_idx(isems[k])
            issue_gather(k)
        return carry

    lax.fori_loop(0, NSTEADY, body, 0)

    for k in range(REM):
        wait_rows(gsems[k])
        wait_idx(jsems[k])
        issue_scatter(k)
        wait_rows(ssems[k])

    eb = base + NFULLA * CA
    pltpu.sync_copy(src_hbm.at[pl.ds(eb, TAILA)], tsrcv)
    pltpu.async_copy(xh_hbm.at[tsrcv], trows, gsem0).wait()
    pltpu.sync_copy(dst_hbm.at[pl.ds(eb, TAILA)], tdstv)
    pltpu.sync_copy(trows, agg_sh.at[tdstv], add=True)
    plsc.subcore_barrier()

    @pl.when(s < NS - 1)
    def _():
        pltpu.sync_copy(agg_sh.at[pl.ds(r0, SLAB)],
                        out_hbm.at[c, pl.ds(r0, SLAB)])

    @pl.when(s == NS - 1)
    def _():
        pltpu.sync_copy(agg_sh.at[pl.ds(r0, SLAB_LAST)],
                        out_hbm.at[c, pl.ds(r0, SLAB_LAST)])


_sc_agg = functools.partial(
    pl.kernel,
    out_type=jax.ShapeDtypeStruct((NC, N, H), jnp.float32),
    mesh=_mesh,
    scratch_types=(
        [pltpu.VMEM_SHARED((N, H), jnp.float32)]
        + [pltpu.VMEM((CA, H), jnp.float32)] * NB
        + [pltpu.VMEM((CA,), jnp.int32)] * (2 * NB)
        + [pltpu.VMEM((TAILA,), jnp.int32)] * 2
        + [pltpu.VMEM((TAILA, H), jnp.float32)]
        + [pltpu.SemaphoreType.DMA] * (4 * NB)
    ),
)(_sc_agg_body)


_R = 1000
_GRID = N // _R


def _mm_body(x_ref, w_ref, b_ref, o_ref):
    o_ref[...] = (jnp.dot(x_ref[...], w_ref[...],
                          preferred_element_type=jnp.float32) + b_ref[...])


def _tc_mm(x, wt, b):
    return pl.pallas_call(
        _mm_body,
        grid=(_GRID,),
        in_specs=[
            pl.BlockSpec((_R, wt.shape[0]), lambda i: (i, 0)),
            pl.BlockSpec(wt.shape, lambda i: (0, 0)),
            pl.BlockSpec((1, wt.shape[1]), lambda i: (0, 0)),
        ],
        out_specs=pl.BlockSpec((_R, wt.shape[1]), lambda i: (i, 0)),
        out_shape=jax.ShapeDtypeStruct((N, wt.shape[1]), jnp.float32),
    )(x, wt, b)


def _post_body(sp_ref, xh_ref, eaggp_ref, degp_ref, wet_ref, be_ref,
               p_ref, st_ref, acc):
    eagg = eaggp_ref[0] + eaggp_ref[1]
    deg = degp_ref[0, :, 0:1] + degp_ref[1, :, 0:1]
    p = (sp_ref[0] + sp_ref[1] + xh_ref[...]
         + jnp.dot(eagg, wet_ref[...], preferred_element_type=jnp.float32)
         + deg * be_ref[...])
    p_ref[...] = p

    @pl.when(pl.program_id(0) == 0)
    def _():
        acc[...] = jnp.zeros_like(acc)

    acc[0:1, :] += jnp.sum(p, axis=0, keepdims=True)
    acc[1:2, :] += jnp.sum(p * p, axis=0, keepdims=True)

    @pl.when(pl.program_id(0) == _GRID - 1)
    def _():
        st_ref[...] = acc[...]


def _tc_post(sp, xh, eaggp, degp, wet, be):
    return pl.pallas_call(
        _post_body,
        grid=(_GRID,),
        in_specs=[
            pl.BlockSpec((NC, _R, H), lambda i: (0, i, 0)),
            pl.BlockSpec((_R, H), lambda i: (i, 0)),
            pl.BlockSpec((NC, _R, ED), lambda i: (0, i, 0)),
            pl.BlockSpec((NC, _R, ED), lambda i: (0, i, 0)),
            pl.BlockSpec((ED, H), lambda i: (0, 0)),
            pl.BlockSpec((1, H), lambda i: (0, 0)),
        ],
        out_specs=[
            pl.BlockSpec((_R, H), lambda i: (i, 0)),
            pl.BlockSpec((2, H), lambda i: (0, 0)),
        ],
        out_shape=[
            jax.ShapeDtypeStruct((N, H), jnp.float32),
            jax.ShapeDtypeStruct((2, H), jnp.float32),
        ],
        scratch_shapes=[pltpu.VMEM((2, H), jnp.float32)],
    )(sp, xh, eaggp, degp, wet, be)


def _bn_mm_body(p_ref, st_ref, g_ref, beta_ref, wt_ref, b_ref, o_ref):
    mu = st_ref[0:1, :] * (1.0 / N)
    var = st_ref[1:2, :] * (1.0 / N) - mu * mu
    xn = (p_ref[...] - mu) * lax.rsqrt(var + EPS) * g_ref[...] + beta_ref[...]
    h = jnp.maximum(xn, 0.0)
    o_ref[...] = (jnp.dot(h, wt_ref[...],
                          preferred_element_type=jnp.float32) + b_ref[...])


def _tc_bn_mm(p, st, g, beta, wt, b):
    return pl.pallas_call(
        _bn_mm_body,
        grid=(_GRID,),
        in_specs=[
            pl.BlockSpec((_R, H), lambda i: (i, 0)),
            pl.BlockSpec((2, H), lambda i: (0, 0)),
            pl.BlockSpec((1, H), lambda i: (0, 0)),
            pl.BlockSpec((1, H), lambda i: (0, 0)),
            pl.BlockSpec((H, H), lambda i: (0, 0)),
            pl.BlockSpec((1, H), lambda i: (0, 0)),
        ],
        out_specs=pl.BlockSpec((_R, H), lambda i: (i, 0)),
        out_shape=jax.ShapeDtypeStruct((N, H), jnp.float32),
    )(p, st, g, beta, wt, b)


def _final_body(p_ref, st_ref, g_ref, beta_ref, batch_ref, wfct_ref, bfc_ref,
                o_ref, accs, accc):
    mu = st_ref[0:1, :] * (1.0 / N)
    var = st_ref[1:2, :] * (1.0 / N) - mu * mu
    xn = (p_ref[...] - mu) * lax.rsqrt(var + EPS) * g_ref[...] + beta_ref[...]
    h = jnp.maximum(xn, 0.0)
    b = batch_ref[0, 0, :]
    oh = (b[:, None] == lax.broadcasted_iota(jnp.int32, (1, G), 1)
          ).astype(jnp.float32)

    @pl.when(pl.program_id(0) == 0)
    def _():
        accs[...] = jnp.zeros_like(accs)
        accc[...] = jnp.zeros_like(accc)

    dn = (((0,), (0,)), ((), ()))
    accs[...] += lax.dot_general(oh, h, dn,
                                 preferred_element_type=jnp.float32)
    accc[...] += lax.dot_general(oh, jnp.ones_like(h), dn,
                                 preferred_element_type=jnp.float32)

    @pl.when(pl.program_id(0) == _GRID - 1)
    def _():
        pooled = accs[...] / jnp.maximum(accc[...], 1.0)
        o_ref[...] = (jnp.dot(pooled, wfct_ref[...],
                              preferred_element_type=jnp.float32)
                      + bfc_ref[...])


def _tc_final(p, st, g, beta, batch3, wfct, bfc):
    return pl.pallas_call(
        _final_body,
        grid=(_GRID,),
        in_specs=[
            pl.BlockSpec((_R, H), lambda i: (i, 0)),
            pl.BlockSpec((2, H), lambda i: (0, 0)),
            pl.BlockSpec((1, H), lambda i: (0, 0)),
            pl.BlockSpec((1, H), lambda i: (0, 0)),
            pl.BlockSpec((1, 1, _R), lambda i: (i, 0, 0)),
            pl.BlockSpec((H, OUT), lambda i: (0, 0)),
            pl.BlockSpec((1, OUT), lambda i: (0, 0)),
        ],
        out_specs=pl.BlockSpec((G, OUT), lambda i: (0, 0)),
        out_shape=jax.ShapeDtypeStruct((G, OUT), jnp.float32),
        scratch_shapes=[pltpu.VMEM((G, H), jnp.float32),
                        pltpu.VMEM((G, H), jnp.float32)],
    )(p, st, g, beta, batch3, wfct, bfc)


def kernel(x, edge_attr, Wn1, bn1, We1, be1, Wn2, bn2, We2, be2,
           Wn3, bn3, We3, be3, g1, beta1, g2, beta2, g3, beta3,
           Wfc, bfc, edge_index, batch):
    f32 = jnp.float32
    src = edge_index[0].astype(jnp.int32)
    dst = edge_index[1].astype(jnp.int32)
    batch3 = batch.astype(jnp.int32).reshape(_GRID, 1, _R)

    zeros128 = jnp.zeros((SLAB, H), f32)
    zeros16 = jnp.zeros((SLAB, ED), f32)
    ones16 = jnp.ones((C, ED), f32)

    def row(v):
        return v.reshape(1, -1).astype(f32)

    eaggp, degp = _sc_pre(edge_attr.astype(f32), dst, zeros16, ones16)

    xh1 = _tc_mm(x.astype(f32), Wn1.T.astype(f32), row(bn1))
    sp1 = _sc_agg(xh1, src, dst, zeros128)
    p1, st1 = _tc_post(sp1, xh1, eaggp, degp, We1.T.astype(f32), row(be1))

    xh2 = _tc_bn_mm(p1, st1, row(g1), row(beta1), Wn2.T.astype(f32), row(bn2))
    sp2 = _sc_agg(xh2, src, dst, zeros128)
    p2, st2 = _tc_post(sp2, xh2, eaggp, degp, We2.T.astype(f32), row(be2))

    xh3 = _tc_bn_mm(p2, st2, row(g2), row(beta2), Wn3.T.astype(f32), row(bn3))
    sp3 = _sc_agg(xh3, src, dst, zeros128)
    p3, st3 = _tc_post(sp3, xh3, eaggp, degp, We3.T.astype(f32), row(be3))

    return _tc_final(p3, st3, row(g3), row(beta3), batch3,
                     Wfc.T.astype(f32), row(bfc))
